# baseline (speedup 1.0000x reference)
import functools

import jax
import jax.numpy as jnp
from jax import lax
from jax.experimental import pallas as pl
from jax.experimental.pallas import tpu as pltpu
from jax.experimental.pallas import tpu_sc as plsc

_EMBED = 64
_BLK = 128


@functools.lru_cache(maxsize=None)
def _make_kernel(J: int, I: int, V: int):
  info = plsc.get_sparse_core_info()
  nc = info.num_cores
  nw = nc * info.num_subcores
  n_blocks = J * (I // _BLK)
  bpw = n_blocks // nw
  ib_count = I // _BLK

  mesh = plsc.VectorSubcoreMesh(core_axis_name="c", subcore_axis_name="s")

  @functools.partial(
      pl.kernel,
      mesh=mesh,
      out_type=jax.ShapeDtypeStruct((J, 8, ib_count, 8, _BLK), jnp.float32),
      scratch_types=[
          pltpu.VMEM((bpw, _BLK), jnp.int32),
          pltpu.VMEM((_BLK, _EMBED), jnp.float32),
          pltpu.VMEM((_BLK, _EMBED), jnp.float32),
          pltpu.VMEM((_EMBED, _BLK), jnp.float32),
          pltpu.VMEM((_EMBED, _BLK), jnp.float32),
          pltpu.SemaphoreType.DMA,
          pltpu.SemaphoreType.DMA,
      ],
      compiler_params=pltpu.CompilerParams(
          use_tc_tiling_on_sc=False, needs_layout_passes=False
      ),
  )
  def k(tok_hbm, table_hbm, out_hbm, idx_v, g0, g1, t0, t1, gsem, osem):
    wid = lax.axis_index("s") * nc + lax.axis_index("c")
    base = wid * bpw
    pltpu.sync_copy(tok_hbm.at[pl.ds(base, bpw)], idx_v)

    kv = lax.iota(jnp.int32, 16)
    cvs = [c0 * 16 + kv for c0 in range(4)]

    def fire_gather(b, g):
      pltpu.async_copy(table_hbm.at[idx_v.at[b]], g, gsem)

    def wait_gather(b, g):
      pltpu.make_async_copy(table_hbm.at[idx_v.at[b]], g, gsem).wait()

    def out_descs(b, t):
      gb = base + b
      j = lax.div(gb, ib_count)
      ib = lax.rem(gb, ib_count)
      return [
          pltpu.make_async_copy(
              t.at[pl.ds(ch * 8, 8), :],
              out_hbm.at[j, ch, ib],
              osem,
          )
          for ch in range(8)
      ]

    def transpose(g, t):
      @pl.loop(0, _BLK, step=2, init_carry=jnp.zeros((16,), jnp.int32),
               unroll=2)
      def _(il, ilv):
        xa = [g[il, pl.ds(c0 * 16, 16)] for c0 in range(4)]
        xb = [g[il + 1, pl.ds(c0 * 16, 16)] for c0 in range(4)]
        ilv1 = ilv + 1
        for c0 in range(4):
          plsc.store_scatter(t, [cvs[c0], ilv], xa[c0])
        for c0 in range(4):
          plsc.store_scatter(t, [cvs[c0], ilv1], xb[c0])
        return ilv + 2

    fire_gather(0, g0)

    @pl.loop(0, bpw // 2)
    def _(p):
      b0 = p * 2
      b1 = b0 + 1

      @pl.when(p > 0)
      def _():
        for d in out_descs(b0 - 2, t0):
          d.wait()

      fire_gather(b1, g1)
      wait_gather(b0, g0)
      transpose(g0, t0)
      for d in out_descs(b0, t0):
        d.start()

      @pl.when(p > 0)
      def _():
        for d in out_descs(b1 - 2, t1):
          d.wait()

      @pl.when(p + 1 < bpw // 2)
      def _():
        fire_gather(b0 + 2, g0)

      wait_gather(b1, g1)
      transpose(g1, t1)
      for d in out_descs(b1, t1):
        d.start()

    for d in out_descs(bpw - 2, t0):
      d.wait()
    for d in out_descs(bpw - 1, t1):
      d.wait()

  return k


def kernel(token, table):
  I, J = token.shape
  V = table.shape[0]
  tok = token.T.reshape(J * I // _BLK, _BLK)
  out5 = _make_kernel(J, I, V)(tok, table)
  return out5.transpose(2, 4, 0, 1, 3).reshape(I, J, _EMBED)


# speedup vs baseline: 1.7905x; 1.7905x over previous
import functools

import jax
import jax.numpy as jnp
from jax import lax
from jax.experimental import pallas as pl
from jax.experimental.pallas import tpu as pltpu
from jax.experimental.pallas import tpu_sc as plsc

_EMBED = 64
_BLK = 128


@functools.lru_cache(maxsize=None)
def _make_kernel(J: int, I: int, V: int):
  info = plsc.get_sparse_core_info()
  nc = info.num_cores
  nw = nc * info.num_subcores
  n_blocks = J * (I // _BLK)
  bpw = n_blocks // nw
  ib_count = I // _BLK

  mesh = plsc.VectorSubcoreMesh(core_axis_name="c", subcore_axis_name="s")

  @functools.partial(
      pl.kernel,
      mesh=mesh,
      out_type=jax.ShapeDtypeStruct((J, 8, ib_count, 8, _BLK), jnp.float32),
      scratch_types=[
          pltpu.VMEM((bpw, _BLK), jnp.int32),
          pltpu.VMEM((_BLK, _EMBED), jnp.float32),
          pltpu.VMEM((_BLK, _EMBED), jnp.float32),
          pltpu.VMEM((_BLK, _EMBED), jnp.float32),
          pltpu.VMEM((_BLK, _EMBED), jnp.float32),
          pltpu.VMEM((_EMBED, _BLK + 1), jnp.float32),
          pltpu.VMEM((_EMBED, _BLK + 1), jnp.float32),
          pltpu.SemaphoreType.DMA,
          pltpu.SemaphoreType.DMA,
      ],
      compiler_params=pltpu.CompilerParams(
          use_tc_tiling_on_sc=False, needs_layout_passes=False
      ),
  )
  def k(tok_hbm, table_hbm, out_hbm, idx_v, g0, g1, g2, g3, t0, t1,
        gsem, osem):
    wid = lax.axis_index("s") * nc + lax.axis_index("c")
    base = wid * bpw
    pltpu.sync_copy(tok_hbm.at[pl.ds(base, bpw)], idx_v)

    kv = lax.iota(jnp.int32, 16)
    cvs = [c0 * 16 + kv for c0 in range(4)]

    def fire_gather(b, g):
      pltpu.async_copy(table_hbm.at[idx_v.at[b]], g, gsem)

    def wait_gather(b, g):
      pltpu.make_async_copy(table_hbm.at[idx_v.at[b]], g, gsem).wait()

    def out_descs(b, t):
      gb = base + b
      j = lax.div(gb, ib_count)
      ib = lax.rem(gb, ib_count)
      return [
          pltpu.make_async_copy(
              t.at[pl.ds(ch * 8, 8), pl.ds(0, _BLK)],
              out_hbm.at[j, ch, ib],
              osem,
          )
          for ch in range(8)
      ]

    def transpose(g, t):
      @pl.loop(0, _BLK, step=2, init_carry=jnp.zeros((16,), jnp.int32),
               unroll=2)
      def _(il, ilv):
        xa = [g[il, pl.ds(c0 * 16, 16)] for c0 in range(4)]
        xb = [g[il + 1, pl.ds(c0 * 16, 16)] for c0 in range(4)]
        ilv1 = ilv + 1
        for c0 in range(4):
          plsc.store_scatter(t, [cvs[c0], ilv], xa[c0])
        for c0 in range(4):
          plsc.store_scatter(t, [cvs[c0], ilv1], xb[c0])
        return ilv + 2

    gs = [g0, g1, g2, g3]
    ts = [t0, t1]
    fire_gather(0, g0)
    fire_gather(1, g1)
    fire_gather(2, g2)

    @pl.loop(0, bpw // 4)
    def _(p):
      for q in range(4):
        b = p * 4 + q

        @pl.when(b + 3 < bpw)
        def _():
          fire_gather(b + 3, gs[(q + 3) % 4])

        wait_gather(b, gs[q])

        if q < 2:
          @pl.when(p > 0)
          def _():
            for d in out_descs(b - 2, ts[q % 2]):
              d.wait()
        else:
          for d in out_descs(b - 2, ts[q % 2]):
            d.wait()

        transpose(gs[q], ts[q % 2])
        for d in out_descs(b, ts[q % 2]):
          d.start()

    for d in out_descs(bpw - 2, t0):
      d.wait()
    for d in out_descs(bpw - 1, t1):
      d.wait()

  return k


def kernel(token, table):
  I, J = token.shape
  V = table.shape[0]
  tok = token.T.reshape(J * I // _BLK, _BLK)
  out5 = _make_kernel(J, I, V)(tok, table)
  return out5.transpose(2, 4, 0, 1, 3).reshape(I, J, _EMBED)
